# baseline (speedup 1.0000x reference)
import functools

import jax
import jax.numpy as jnp
from jax import lax
from jax.experimental import pallas as pl
from jax.experimental.pallas import tpu as pltpu
from jax.experimental.pallas import tpu_sc as plsc

NUM_USER = 1000000
NUM_MOVIE = 1000000
EMB = 32
BATCH = 16384

_NC = 2
_NS = 16
_NW = _NC * _NS
_BPW = BATCH // _NW
_L = 16
_ICH = 128
_NCH = _BPW // _ICH


def _mf_body(u_hbm, v_hbm, ue_hbm, me_hbm, ub_hbm, mb_hbm, out_hbm,
             idx_u, idx_v, gix_u, gix_v, ucols, vcols, bu_v, bv_v,
             out_v, sem, bsem):
    wid = lax.axis_index("s") * _NC + lax.axis_index("c")
    base = wid * _BPW

    pltpu.sync_copy(u_hbm.at[pl.ds(base, _BPW)], idx_u)
    pltpu.sync_copy(v_hbm.at[pl.ds(base, _BPW)], idx_v)

    b1 = pltpu.async_copy(ub_hbm.at[idx_u], bu_v, bsem)
    b2 = pltpu.async_copy(mb_hbm.at[idx_v], bv_v, bsem)

    def build(j, _):
        s = pl.ds(j * _L, _L)
        uvec = idx_u[s]
        vvec = idx_v[s]
        for c in range(EMB):
            t = pl.ds(c * _BPW + j * _L, _L)
            gix_u[t] = uvec + (c * NUM_USER)
            gix_v[t] = vvec + (c * NUM_MOVIE)
        return _
    lax.fori_loop(0, _BPW // _L, build, None)

    copies = []
    for c in range(EMB):
        for j in range(_NCH):
            s = pl.ds(c * _BPW + j * _ICH, _ICH)
            copies.append(pltpu.async_copy(ue_hbm.at[gix_u.at[s]],
                                           ucols.at[s], sem))
            copies.append(pltpu.async_copy(me_hbm.at[gix_v.at[s]],
                                           vcols.at[s], sem))
    for c in copies:
        c.wait()
    b1.wait()
    b2.wait()

    def chunk(i, _):
        r0 = i * _L
        acc = bu_v[pl.ds(r0, _L)] + bv_v[pl.ds(r0, _L)]
        for c in range(EMB):
            o = c * _BPW + r0
            acc = acc + ucols[pl.ds(o, _L)] * vcols[pl.ds(o, _L)]
        out_v[pl.ds(r0, _L)] = 4.0 / (1.0 + jnp.exp(-acc)) + 1.0
        return _

    lax.fori_loop(0, _BPW // _L, chunk, None)

    pltpu.sync_copy(out_v, out_hbm.at[pl.ds(base, _BPW)])


@jax.jit
def _mf_call(u, v, ue_flat, me_flat, ub_flat, mb_flat):
    mesh = plsc.VectorSubcoreMesh(core_axis_name="c", subcore_axis_name="s")
    return pl.kernel(
        _mf_body,
        out_type=jax.ShapeDtypeStruct((BATCH,), jnp.float32),
        mesh=mesh,
        compiler_params=pltpu.CompilerParams(
            needs_layout_passes=False, use_tc_tiling_on_sc=False),
        scratch_types=[
            pltpu.VMEM((_BPW,), jnp.int32),
            pltpu.VMEM((_BPW,), jnp.int32),
            pltpu.VMEM((EMB * _BPW,), jnp.int32),
            pltpu.VMEM((EMB * _BPW,), jnp.int32),
            pltpu.VMEM((EMB * _BPW,), jnp.float32),
            pltpu.VMEM((EMB * _BPW,), jnp.float32),
            pltpu.VMEM((_BPW,), jnp.float32),
            pltpu.VMEM((_BPW,), jnp.float32),
            pltpu.VMEM((_BPW,), jnp.float32),
            pltpu.SemaphoreType.DMA,
            pltpu.SemaphoreType.DMA,
        ],
    )(u, v, ue_flat, me_flat, ub_flat, mb_flat)


def kernel(u, v, user_emb, movie_emb, user_bias, movie_bias):
    u = u.astype(jnp.int32)
    v = v.astype(jnp.int32)
    return _mf_call(u, v, user_emb.T.reshape(-1), movie_emb.T.reshape(-1),
                    user_bias.reshape(-1), movie_bias.reshape(-1))


# speedup vs baseline: 1.0006x; 1.0006x over previous
import functools

import jax
import jax.numpy as jnp
from jax import lax
from jax.experimental import pallas as pl
from jax.experimental.pallas import tpu as pltpu
from jax.experimental.pallas import tpu_sc as plsc

NUM_USER = 1000000
NUM_MOVIE = 1000000
EMB = 32
BATCH = 16384

_NC = 2
_NS = 16
_NW = _NC * _NS
_BPW = BATCH // _NW
_L = 16
_ICH = 128
_NCH = _BPW // _ICH


def _mf_body(u_hbm, v_hbm, ue_hbm, me_hbm, ub_hbm, mb_hbm, out_hbm,
             idx_u, idx_v, ucols, vcols, bu_v, bv_v,
             out_v, sem, bsem):
    wid = lax.axis_index("s") * _NC + lax.axis_index("c")
    base = wid * _BPW

    pltpu.sync_copy(u_hbm.at[pl.ds(base, _BPW)], idx_u)
    pltpu.sync_copy(v_hbm.at[pl.ds(base, _BPW)], idx_v)

    b1 = pltpu.async_copy(ub_hbm.at[idx_u], bu_v, bsem)
    b2 = pltpu.async_copy(mb_hbm.at[idx_v], bv_v, bsem)

    copies = []
    for c in range(EMB):
        for j in range(_NCH):
            s = pl.ds(j * _ICH, _ICH)
            t = pl.ds(c * _BPW + j * _ICH, _ICH)
            copies.append(pltpu.async_copy(ue_hbm.at[c].at[idx_u.at[s]],
                                           ucols.at[t], sem))
            copies.append(pltpu.async_copy(me_hbm.at[c].at[idx_v.at[s]],
                                           vcols.at[t], sem))
    for c in copies:
        c.wait()
    b1.wait()
    b2.wait()

    def chunk(i, _):
        r0 = i * _L
        acc = bu_v[pl.ds(r0, _L)] + bv_v[pl.ds(r0, _L)]
        for c in range(EMB):
            o = c * _BPW + r0
            acc = acc + ucols[pl.ds(o, _L)] * vcols[pl.ds(o, _L)]
        out_v[pl.ds(r0, _L)] = 4.0 / (1.0 + jnp.exp(-acc)) + 1.0
        return _

    lax.fori_loop(0, _BPW // _L, chunk, None)

    pltpu.sync_copy(out_v, out_hbm.at[pl.ds(base, _BPW)])


@jax.jit
def _mf_call(u, v, ue_flat, me_flat, ub_flat, mb_flat):
    mesh = plsc.VectorSubcoreMesh(core_axis_name="c", subcore_axis_name="s")
    return pl.kernel(
        _mf_body,
        out_type=jax.ShapeDtypeStruct((BATCH,), jnp.float32),
        mesh=mesh,
        compiler_params=pltpu.CompilerParams(
            needs_layout_passes=False, use_tc_tiling_on_sc=False),
        scratch_types=[
            pltpu.VMEM((_BPW,), jnp.int32),
            pltpu.VMEM((_BPW,), jnp.int32),
            pltpu.VMEM((EMB * _BPW,), jnp.float32),
            pltpu.VMEM((EMB * _BPW,), jnp.float32),
            pltpu.VMEM((_BPW,), jnp.float32),
            pltpu.VMEM((_BPW,), jnp.float32),
            pltpu.VMEM((_BPW,), jnp.float32),
            pltpu.SemaphoreType.DMA,
            pltpu.SemaphoreType.DMA,
        ],
    )(u, v, ue_flat, me_flat, ub_flat, mb_flat)


def kernel(u, v, user_emb, movie_emb, user_bias, movie_bias):
    u = u.astype(jnp.int32)
    v = v.astype(jnp.int32)
    return _mf_call(u, v, user_emb.T, movie_emb.T,
                    user_bias.reshape(-1), movie_bias.reshape(-1))


# speedup vs baseline: 5.7170x; 5.7134x over previous
import functools

import jax
import jax.numpy as jnp
from jax import lax
from jax.experimental import pallas as pl
from jax.experimental.pallas import tpu as pltpu
from jax.experimental.pallas import tpu_sc as plsc

NUM_USER = 1000000
NUM_MOVIE = 1000000
EMB = 32
BATCH = 16384

_NC = 2
_NS = 16
_NW = _NC * _NS
_BPW = BATCH // _NW
_ICH = 128
_NCH = _BPW // _ICH
_L = 16


def _mf_body(u_hbm, v_hbm, ue_hbm, me_hbm, ub_hbm, mb_hbm, out_hbm,
             idx_u, idx_v, urows, vrows, bu_v, bv_v, out_v, sem):
    wid = lax.axis_index("s") * _NC + lax.axis_index("c")
    base = wid * _BPW

    pltpu.sync_copy(u_hbm.at[pl.ds(base, _BPW)], idx_u)
    pltpu.sync_copy(v_hbm.at[pl.ds(base, _BPW)], idx_v)

    copies = []
    for j in range(_NCH):
        s = pl.ds(j * _ICH, _ICH)
        copies.append(pltpu.async_copy(ue_hbm.at[idx_u.at[s]], urows.at[s], sem))
        copies.append(pltpu.async_copy(me_hbm.at[idx_v.at[s]], vrows.at[s], sem))
        copies.append(pltpu.async_copy(ub_hbm.at[idx_u.at[s]], bu_v.at[s], sem))
        copies.append(pltpu.async_copy(mb_hbm.at[idx_v.at[s]], bv_v.at[s], sem))
    for c in copies:
        c.wait()

    iota = lax.iota(jnp.int32, _L)

    def chunk(i, _):
        r0 = i * _L
        rows = r0 + iota
        acc = bu_v[pl.ds(r0, _L)] + bv_v[pl.ds(r0, _L)]
        for d in range(EMB):
            dcol = jnp.full((_L,), d, jnp.int32)
            uc = plsc.load_gather(urows, [rows, dcol])
            vc = plsc.load_gather(vrows, [rows, dcol])
            acc = acc + uc * vc
        out_v[pl.ds(r0, _L)] = 4.0 / (1.0 + jnp.exp(-acc)) + 1.0
        return _

    lax.fori_loop(0, _BPW // _L, chunk, None)

    pltpu.sync_copy(out_v, out_hbm.at[pl.ds(base, _BPW)])


@jax.jit
def _mf_call(u, v, user_emb, movie_emb, user_bias, movie_bias):
    mesh = plsc.VectorSubcoreMesh(core_axis_name="c", subcore_axis_name="s")
    return pl.kernel(
        _mf_body,
        out_type=jax.ShapeDtypeStruct((BATCH,), jnp.float32),
        mesh=mesh,
        compiler_params=pltpu.CompilerParams(
            needs_layout_passes=False, use_tc_tiling_on_sc=False),
        scratch_types=[
            pltpu.VMEM((_BPW,), jnp.int32),
            pltpu.VMEM((_BPW,), jnp.int32),
            pltpu.VMEM((_BPW, EMB), jnp.float32),
            pltpu.VMEM((_BPW, EMB), jnp.float32),
            pltpu.VMEM((_BPW,), jnp.float32),
            pltpu.VMEM((_BPW,), jnp.float32),
            pltpu.VMEM((_BPW,), jnp.float32),
            pltpu.SemaphoreType.DMA,
        ],
    )(u, v, user_emb, movie_emb, user_bias, movie_bias)


def kernel(u, v, user_emb, movie_emb, user_bias, movie_bias):
    u = u.astype(jnp.int32)
    v = v.astype(jnp.int32)
    return _mf_call(u, v, user_emb, movie_emb,
                    user_bias.reshape(-1), movie_bias.reshape(-1))
